# baseline (speedup 1.0000x reference)
import functools

import jax
import jax.numpy as jnp
from jax.experimental import pallas as pl

ALPHA = 0.3
BETA = 0.7
GAMMA = 1.33
SMOOTH = 1e-06
LOVASZ_WEIGHT = 0.2

_R = 2048
_C = 128


def _elem_kernel(x_ref, t_ref, err_ref, tp_ref, fn_ref, fp_ref):
    x = x_ref[0]
    t = t_ref[0].astype(jnp.float32)
    p = jax.nn.sigmoid(x)
    tp_ref[0, 0, :] = jnp.sum(p * t, axis=0)
    fn_ref[0, 0, :] = jnp.sum((1.0 - p) * t, axis=0)
    fp_ref[0, 0, :] = jnp.sum(p * (1.0 - t), axis=0)
    err_ref[0] = 1.0 - x * (2.0 * t - 1.0)


def _lovasz_flat(errors, labels):
    perm = jnp.argsort(-errors)
    errors_sorted = errors[perm]
    labels_sorted = labels[perm]
    p = labels_sorted.sum()
    n = labels_sorted.shape[0] - p
    intersection = p - jnp.cumsum(labels_sorted)
    union = p + jnp.cumsum(1.0 - labels_sorted)
    jaccard = 1.0 - intersection / union
    jaccard_diff = jnp.concatenate([jaccard[:1], jaccard[1:] - jaccard[:-1]])
    grad = jnp.where(n > 0, jaccard_diff, jaccard)
    return jnp.dot(jax.nn.relu(errors_sorted), grad)


def kernel(logits, targets):
    B = logits.shape[0]
    x = logits.reshape(B, _R, _C)
    t = targets.reshape(B, _R, _C)
    errs, tp, fn, fp = pl.pallas_call(
        _elem_kernel,
        grid=(B,),
        in_specs=[
            pl.BlockSpec((1, _R, _C), lambda i: (i, 0, 0)),
            pl.BlockSpec((1, _R, _C), lambda i: (i, 0, 0)),
        ],
        out_specs=[
            pl.BlockSpec((1, _R, _C), lambda i: (i, 0, 0)),
            pl.BlockSpec((1, 1, _C), lambda i: (i, 0, 0)),
            pl.BlockSpec((1, 1, _C), lambda i: (i, 0, 0)),
            pl.BlockSpec((1, 1, _C), lambda i: (i, 0, 0)),
        ],
        out_shape=[
            jax.ShapeDtypeStruct((B, _R, _C), jnp.float32),
            jax.ShapeDtypeStruct((B, 1, _C), jnp.float32),
            jax.ShapeDtypeStruct((B, 1, _C), jnp.float32),
            jax.ShapeDtypeStruct((B, 1, _C), jnp.float32),
        ],
    )(x, t)
    tp = tp.reshape(B, _C).sum(axis=1)
    fn = fn.reshape(B, _C).sum(axis=1)
    fp = fp.reshape(B, _C).sum(axis=1)
    tversky = (tp + SMOOTH) / (tp + ALPHA * fn + BETA * fp + SMOOTH)
    ft = jnp.mean((1.0 - tversky) ** GAMMA)
    lab = targets.reshape(B, -1).astype(jnp.float32)
    lov = jnp.mean(jax.vmap(_lovasz_flat)(errs.reshape(B, -1), lab))
    return ft + LOVASZ_WEIGHT * lov
